# baseline (speedup 1.0000x reference)
import functools

import jax
import jax.numpy as jnp
from jax import lax
from jax.experimental import pallas as pl
from jax.experimental.pallas import tpu as pltpu
from jax.experimental.pallas import tpu_sc as plsc

B = 1024
L = 50
LP = 64
C = 32
BT = B * L

def _dp_body(x_ref, y_ref, m_ref, acc_ref):
    srow = lax.broadcasted_iota(jnp.int32, (LP, B), 0)
    x = x_ref[...]
    m_ref[...] = jnp.zeros((LP, B), jnp.int32)
    acc_ref[...] = jnp.full((LP, B), -1, jnp.int32)

    def step(i, carry):
        M = m_ref[...]
        xi = jnp.sum(jnp.where(srow == i, x, 0), axis=0, keepdims=True)
        eq = x == xi
        shifted = jnp.where(srow == 0, 0, pltpu.roll(M, 1, 0))
        mcur = jnp.where(eq, shifted + 1, 0)
        mm = jnp.where(srow < i, mcur, 0)
        lmax = jnp.max(mm, axis=0, keepdims=True)
        jb = jnp.max(jnp.where(mm == lmax, srow, -1), axis=0, keepdims=True)
        nxt = jnp.sum(jnp.where(srow == jb + 1, x, 0), axis=0, keepdims=True)
        yi = jnp.where(lmax > 0, nxt, -1)
        m_ref[...] = mcur
        acc_ref[...] = jnp.where(srow == i, yi, acc_ref[...])
        return carry

    lax.fori_loop(0, L, step, 0)
    y_ref[...] = acc_ref[...].T


def _run_dp(x_pad):
    return pl.pallas_call(
        _dp_body,
        out_shape=jax.ShapeDtypeStruct((B, LP), jnp.int32),
        scratch_shapes=[pltpu.VMEM((LP, B), jnp.int32),
                        pltpu.VMEM((LP, B), jnp.int32)],
    )(x_pad)


_NW = 32
_BPW = BT // _NW
_RPW = B // _NW
_CHUNK = 80
_NCH = _BPW // _CHUNK
_ZR = 80
_CAP = _BPW + _CHUNK + 16


def _bcast_lane(v, lane_idx):
    return lax.gather(
        v, lane_idx[:, None],
        lax.GatherDimensionNumbers(
            offset_dims=(), collapsed_slice_dims=(0,), start_index_map=(0,)),
        (1,), mode=lax.GatherScatterMode.PROMISE_IN_BOUNDS)


def _bcast0(v):
    return _bcast_lane(v, jnp.zeros((16,), jnp.int32))


def _prefix_incl(v, lanes):
    x = v
    for d in (1, 2, 4, 8):
        sh = _bcast_lane(x, jnp.maximum(lanes - d, 0))
        x = x + jnp.where(lanes >= d, sh, 0)
    return x


def _sc_body(table_hbm, t_hbm, out_hbm,
             t_v, cidx, cdst, cdst2, grp, zbuf, gsem, ssem):
    wid = lax.axis_index("s") * 2 + lax.axis_index("c")
    base = wid * _BPW
    pltpu.sync_copy(t_hbm.at[pl.ds(wid * _RPW, _RPW)], t_v)

    def zstore(j, carry):
        zbuf[j >> 1, pl.ds((j & 1) * 16, 16)] = jnp.zeros((16,), jnp.float32)
        return carry
    lax.fori_loop(0, _ZR * 2, zstore, 0)

    def zfill(j, carry):
        pltpu.sync_copy(zbuf, out_hbm.at[pl.ds(base + j * _ZR, _ZR)])
        return carry
    lax.fori_loop(0, _BPW // _ZR, zfill, 0)

    lanes = lax.broadcasted_iota(jnp.int32, (16,), 0)

    def cgroup(k, n):
        r = k >> 2
        col = (k & 3) * 16
        tv = t_v[r, pl.ds(col, 16)]
        dbase = base + r * L + col
        for j in range(16):
            e = tv[j]
            act = e >= 0

            @pl.when(act)
            def _():
                cidx[pl.ds(n, 16)] = jnp.full((16,), e, jnp.int32)
                cdst[pl.ds(n, 16)] = jnp.full((16,), dbase + j, jnp.int32)

            n = jnp.where(act, n + 1, n)
        return n

    n = lax.fori_loop(0, (_RPW * LP) // 16, cgroup, jnp.int32(0))

    c0 = _bcast0(cidx[pl.ds(0, 16)])
    d0 = _bcast0(cdst[pl.ds(0, 16)])
    for mm in range(_CHUNK // 16):
        cidx[pl.ds(n + mm * 16, 16)] = c0
        cdst[pl.ds(n + mm * 16, 16)] = d0

    def stage(j, carry):
        cdst2[j // 5, pl.ds((j % 5) * 16, 16)] = cdst[pl.ds(j * 16, 16)]
        return carry
    lax.fori_loop(0, (_NCH + 1) * 5, stage, 0)

    nch = (n + (_CHUNK - 1)) // _CHUNK

    def chunk(j, carry):
        g = pltpu.make_async_copy(
            table_hbm.at[cidx.at[pl.ds(j * _CHUNK, _CHUNK)]], grp, gsem)
        g.start()
        g.wait()
        s = pltpu.make_async_copy(grp, out_hbm.at[cdst2.at[j]], ssem)
        s.start()
        s.wait()
        return carry

    lax.fori_loop(0, nch, chunk, 0)


def _run_lookup(emb_weight, t_nw):
    mesh = plsc.VectorSubcoreMesh(core_axis_name="c", subcore_axis_name="s")
    k = functools.partial(
        pl.kernel,
        out_type=jax.ShapeDtypeStruct((BT, C), jnp.float32),
        mesh=mesh,
        scratch_types=[
            pltpu.VMEM((_RPW, LP), jnp.int32),
            pltpu.VMEM((_CAP,), jnp.int32),
            pltpu.VMEM((_CAP,), jnp.int32),
            pltpu.VMEM((_NCH + 1, _CHUNK), jnp.int32),
            pltpu.VMEM((_CHUNK, C), jnp.float32),
            pltpu.VMEM((_ZR, C), jnp.float32),
            pltpu.SemaphoreType.DMA,
            pltpu.SemaphoreType.DMA,
        ],
        compiler_params=pltpu.CompilerParams(use_tc_tiling_on_sc=False),
    )(_sc_body)
    return k(emb_weight, t_nw)


def kernel(idx, emb_weight):
    xt = jnp.pad(idx.T, ((0, LP - L), (0, 0)), constant_values=-1)
    y = _run_dp(xt)
    out = _run_lookup(emb_weight, y)
    return out.reshape(B, L, C)


# speedup vs baseline: 1.0052x; 1.0052x over previous
import functools

import jax
import jax.numpy as jnp
from jax import lax
from jax.experimental import pallas as pl
from jax.experimental.pallas import tpu as pltpu
from jax.experimental.pallas import tpu_sc as plsc

B = 1024
L = 50
LP = 64
C = 32
BT = B * L

def _dp_body(x_ref, y_ref, m_ref, acc_ref):
    srow = lax.broadcasted_iota(jnp.int32, (LP, B), 0)
    x = x_ref[...]
    m_ref[...] = jnp.zeros((LP, B), jnp.int32)
    acc_ref[...] = jnp.full((LP, B), -1, jnp.int32)

    def step(i, carry):
        M = m_ref[...]
        xi = jnp.sum(jnp.where(srow == i, x, 0), axis=0, keepdims=True)
        eq = x == xi
        shifted = jnp.where(srow == 0, 0, pltpu.roll(M, 1, 0))
        mcur = jnp.where(eq, shifted + 1, 0)
        mm = jnp.where(srow < i, mcur, 0)
        lmax = jnp.max(mm, axis=0, keepdims=True)
        jb = jnp.max(jnp.where(mm == lmax, srow, -1), axis=0, keepdims=True)
        nxt = jnp.sum(jnp.where(srow == jb + 1, x, 0), axis=0, keepdims=True)
        yi = jnp.where(lmax > 0, nxt, -1)
        m_ref[...] = mcur
        acc_ref[...] = jnp.where(srow == i, yi, acc_ref[...])
        return carry

    lax.fori_loop(0, L, step, 0)
    yt = acc_ref[...].T
    y_ref[...] = jnp.concatenate([yt[:512], yt[512:]], axis=1)


def _run_dp(x_pad):
    return pl.pallas_call(
        _dp_body,
        out_shape=jax.ShapeDtypeStruct((512, 128), jnp.int32),
        scratch_shapes=[pltpu.VMEM((LP, B), jnp.int32),
                        pltpu.VMEM((LP, B), jnp.int32)],
    )(x_pad)


_NW = 32
_BPW = BT // _NW
_RPW = B // _NW
_CHUNK = 80
_NCH = _BPW // _CHUNK
_ZR = 80
_CAP = _BPW + _CHUNK + 16


def _bcast_lane(v, lane_idx):
    return lax.gather(
        v, lane_idx[:, None],
        lax.GatherDimensionNumbers(
            offset_dims=(), collapsed_slice_dims=(0,), start_index_map=(0,)),
        (1,), mode=lax.GatherScatterMode.PROMISE_IN_BOUNDS)


def _bcast0(v):
    return _bcast_lane(v, jnp.zeros((16,), jnp.int32))


def _prefix_incl(v, lanes):
    x = v
    for d in (1, 2, 4, 8):
        sh = _bcast_lane(x, jnp.maximum(lanes - d, 0))
        x = x + jnp.where(lanes >= d, sh, 0)
    return x


def _sc_body(table_hbm, t_hbm, out_hbm,
             t_v, cidx, cdst, cdst2, grp, zbuf, gsem, ssem):
    wid = lax.axis_index("s") * 2 + lax.axis_index("c")
    base = wid * _BPW
    pltpu.sync_copy(t_hbm.at[pl.ds(wid * 16, 16)], t_v)

    def zstore(j, carry):
        zbuf[j >> 1, pl.ds((j & 1) * 16, 16)] = jnp.zeros((16,), jnp.float32)
        return carry
    lax.fori_loop(0, _ZR * 2, zstore, 0)

    zb0 = wid * 800
    zb1 = 25600 + wid * 800

    def zfill(j, carry):
        pltpu.sync_copy(zbuf, out_hbm.at[pl.ds(zb0 + j * _ZR, _ZR)])
        pltpu.sync_copy(zbuf, out_hbm.at[pl.ds(zb1 + j * _ZR, _ZR)])
        return carry
    lax.fori_loop(0, 800 // _ZR, zfill, 0)

    lanes = lax.broadcasted_iota(jnp.int32, (16,), 0)

    def cgroup(k, n):
        r = k >> 3
        kc = k & 7
        col = kc * 16
        tv = t_v[r, pl.ds(col, 16)]
        b = (kc >> 2) * 512 + wid * 16 + r
        dbase = b * L + (kc & 3) * 16
        for j in range(16):
            e = tv[j]
            act = e >= 0

            @pl.when(act)
            def _():
                cidx[pl.ds(n, 16)] = jnp.full((16,), e, jnp.int32)
                cdst[pl.ds(n, 16)] = jnp.full((16,), dbase + j, jnp.int32)

            n = jnp.where(act, n + 1, n)
        return n

    n = lax.fori_loop(0, 128, cgroup, jnp.int32(0))

    c0 = _bcast0(cidx[pl.ds(0, 16)])
    d0 = _bcast0(cdst[pl.ds(0, 16)])
    for mm in range(_CHUNK // 16):
        cidx[pl.ds(n + mm * 16, 16)] = c0
        cdst[pl.ds(n + mm * 16, 16)] = d0

    def stage(j, carry):
        cdst2[j // 5, pl.ds((j % 5) * 16, 16)] = cdst[pl.ds(j * 16, 16)]
        return carry
    lax.fori_loop(0, (_NCH + 1) * 5, stage, 0)

    nch = (n + (_CHUNK - 1)) // _CHUNK

    def chunk(j, carry):
        g = pltpu.make_async_copy(
            table_hbm.at[cidx.at[pl.ds(j * _CHUNK, _CHUNK)]], grp, gsem)
        g.start()
        g.wait()
        s = pltpu.make_async_copy(grp, out_hbm.at[cdst2.at[j]], ssem)
        s.start()
        s.wait()
        return carry

    lax.fori_loop(0, nch, chunk, 0)


def _run_lookup(emb_weight, t_nw):
    mesh = plsc.VectorSubcoreMesh(core_axis_name="c", subcore_axis_name="s")
    k = functools.partial(
        pl.kernel,
        out_type=jax.ShapeDtypeStruct((BT, C), jnp.float32),
        mesh=mesh,
        scratch_types=[
            pltpu.VMEM((16, 128), jnp.int32),
            pltpu.VMEM((_CAP,), jnp.int32),
            pltpu.VMEM((_CAP,), jnp.int32),
            pltpu.VMEM((_NCH + 1, _CHUNK), jnp.int32),
            pltpu.VMEM((_CHUNK, C), jnp.float32),
            pltpu.VMEM((_ZR, C), jnp.float32),
            pltpu.SemaphoreType.DMA,
            pltpu.SemaphoreType.DMA,
        ],
        compiler_params=pltpu.CompilerParams(use_tc_tiling_on_sc=False),
    )(_sc_body)
    return k(emb_weight, t_nw)


def kernel(idx, emb_weight):
    xt = jnp.pad(idx.T, ((0, LP - L), (0, 0)), constant_values=-1)
    y = _run_dp(xt)
    out = _run_lookup(emb_weight, y)
    return out.reshape(B, L, C)


# speedup vs baseline: 1.0431x; 1.0377x over previous
import functools

import jax
import jax.numpy as jnp
from jax import lax
from jax.experimental import pallas as pl
from jax.experimental.pallas import tpu as pltpu
from jax.experimental.pallas import tpu_sc as plsc

B = 1024
L = 50
LP = 64
C = 32
BT = B * L

def _dp_body(x_ref, y_ref, m_ref, acc_ref):
    srow = lax.broadcasted_iota(jnp.int32, (LP, B), 0)
    x = x_ref[...]
    m_ref[...] = jnp.zeros((LP, B), jnp.int32)
    acc_ref[...] = jnp.full((LP, B), -1, jnp.int32)

    def step(i, carry):
        M = m_ref[...]
        xi = jnp.sum(jnp.where(srow == i, x, 0), axis=0, keepdims=True)
        eq = x == xi
        shifted = jnp.where(srow == 0, 0, pltpu.roll(M, 1, 0))
        mcur = jnp.where(eq, shifted + 1, 0)
        mm = jnp.where(srow < i, mcur, 0)
        lmax = jnp.max(mm, axis=0, keepdims=True)
        jb = jnp.max(jnp.where(mm == lmax, srow, -1), axis=0, keepdims=True)
        nxt = jnp.sum(jnp.where(srow == jb + 1, x, 0), axis=0, keepdims=True)
        yi = jnp.where(lmax > 0, nxt, -1)
        m_ref[...] = mcur
        acc_ref[...] = jnp.where(srow == i, yi, acc_ref[...])
        return carry

    lax.fori_loop(0, L, step, 0)
    yt = acc_ref[...].T
    y_ref[...] = jnp.concatenate([yt[:512], yt[512:]], axis=1)


def _run_dp(x_pad):
    return pl.pallas_call(
        _dp_body,
        out_shape=jax.ShapeDtypeStruct((512, 128), jnp.int32),
        scratch_shapes=[pltpu.VMEM((LP, B), jnp.int32),
                        pltpu.VMEM((LP, B), jnp.int32)],
    )(x_pad)


_NW = 32
_BPW = BT // _NW
_RPW = B // _NW
_CHUNK = 80
_NCH = _BPW // _CHUNK
_ZR = 80
_CAP = _BPW + _CHUNK + 16


def _bcast_lane(v, lane_idx):
    return lax.gather(
        v, lane_idx[:, None],
        lax.GatherDimensionNumbers(
            offset_dims=(), collapsed_slice_dims=(0,), start_index_map=(0,)),
        (1,), mode=lax.GatherScatterMode.PROMISE_IN_BOUNDS)


def _bcast0(v):
    return _bcast_lane(v, jnp.zeros((16,), jnp.int32))


def _prefix_incl(v, lanes):
    x = v
    for d in (1, 2, 4, 8):
        sh = _bcast_lane(x, jnp.maximum(lanes - d, 0))
        x = x + jnp.where(lanes >= d, sh, 0)
    return x


def _sc_body(table_hbm, t_hbm, out_hbm, t_v, cidx, cdst, grp, lsl, gsem, osem):
    wid = lax.axis_index("s") * 2 + lax.axis_index("c")
    pltpu.sync_copy(t_hbm.at[pl.ds(wid * 16, 16)], t_v)

    def zstore(j, carry):
        lsl[j >> 3, pl.ds((j & 7) * 16, 16)] = jnp.zeros((16,), jnp.float32)
        return carry
    lax.fori_loop(0, 400 * 8, zstore, 0)

    def cgroup(k, n):
        r = k >> 3
        kc = k & 7
        tv = t_v[r, pl.ds(kc * 16, 16)]
        ldbase = (kc >> 2) * 800 + r * L + (kc & 3) * 16
        for j in range(16):
            e = tv[j]
            act = e >= 0

            @pl.when(act)
            def _():
                cidx[pl.ds(n, 16)] = jnp.full((16,), e >> 2, jnp.int32)
                cdst[pl.ds(n, 16)] = jnp.full(
                    (16,), ((e & 3) << 16) | (ldbase + j), jnp.int32)

            n = jnp.where(act, n + 1, n)
        return n

    n = lax.fori_loop(0, 128, cgroup, jnp.int32(0))

    c0 = _bcast0(cidx[pl.ds(0, 16)])
    d0 = _bcast0(cdst[pl.ds(0, 16)])
    for mm in range(_CHUNK // 16):
        cidx[pl.ds(n + mm * 16, 16)] = c0
        cdst[pl.ds(n + mm * 16, 16)] = d0

    nch = (n + (_CHUNK - 1)) // _CHUNK

    def chunk(q, carry):
        g = pltpu.make_async_copy(
            table_hbm.at[cidx.at[pl.ds(q * _CHUNK, _CHUNK)]], grp, gsem)
        g.start()
        g.wait()

        def place(v, carry2):
            dv = cdst[pl.ds(q * _CHUNK + v * 16, 16)]
            for j in range(16):
                d = dv[j]
                sub = (d >> 16) * 32
                slot = d & 0xFFFF
                row = slot >> 2
                col = (slot & 3) * 32
                lsl[row, pl.ds(col, 16)] = grp[v * 16 + j, pl.ds(sub, 16)]
                lsl[row, pl.ds(col + 16, 16)] = grp[v * 16 + j,
                                                    pl.ds(sub + 16, 16)]
            return carry2

        lax.fori_loop(0, _CHUNK // 16, place, 0)
        return carry

    lax.fori_loop(0, nch, chunk, 0)

    o0 = pltpu.make_async_copy(
        lsl.at[pl.ds(0, 200)], out_hbm.at[pl.ds(wid * 200, 200)], osem)
    o1 = pltpu.make_async_copy(
        lsl.at[pl.ds(200, 200)], out_hbm.at[pl.ds(6400 + wid * 200, 200)], osem)
    o0.start()
    o1.start()
    o0.wait()
    o1.wait()


def _run_lookup(emb_weight, t_lin):
    mesh = plsc.VectorSubcoreMesh(core_axis_name="c", subcore_axis_name="s")
    k = functools.partial(
        pl.kernel,
        out_type=jax.ShapeDtypeStruct((12800, 128), jnp.float32),
        mesh=mesh,
        scratch_types=[
            pltpu.VMEM((16, 128), jnp.int32),
            pltpu.VMEM((_CAP,), jnp.int32),
            pltpu.VMEM((_CAP,), jnp.int32),
            pltpu.VMEM((_CHUNK, 128), jnp.float32),
            pltpu.VMEM((400, 128), jnp.float32),
            pltpu.SemaphoreType.DMA,
            pltpu.SemaphoreType.DMA,
        ],
        compiler_params=pltpu.CompilerParams(use_tc_tiling_on_sc=True),
    )(_sc_body)
    return k(emb_weight.reshape(250000, 128), t_lin)


def kernel(idx, emb_weight):
    xt = jnp.pad(idx.T, ((0, LP - L), (0, 0)), constant_values=-1)
    y = _run_dp(xt)
    out = _run_lookup(emb_weight, y)
    return out.reshape(BT, C).reshape(B, L, C)


# speedup vs baseline: 1.0432x; 1.0001x over previous
import functools

import jax
import jax.numpy as jnp
from jax import lax
from jax.experimental import pallas as pl
from jax.experimental.pallas import tpu as pltpu
from jax.experimental.pallas import tpu_sc as plsc

B = 1024
L = 50
LP = 64
C = 32
BT = B * L

def _dp_body(x_ref, y_ref, m_ref, acc_ref):
    srow = lax.broadcasted_iota(jnp.int32, (LP, B), 0)
    x = x_ref[...]
    m_ref[...] = jnp.zeros((LP, B), jnp.int32)
    acc_ref[...] = jnp.full((LP, B), -1, jnp.int32)

    def step(i, carry):
        M = m_ref[...]
        xi = jnp.sum(jnp.where(srow == i, x, 0), axis=0, keepdims=True)
        eq = x == xi
        shifted = jnp.where(srow == 0, 0, pltpu.roll(M, 1, 0))
        mcur = jnp.where(eq, shifted + 1, 0)
        mm = jnp.where(srow < i, mcur, 0)
        lmax = jnp.max(mm, axis=0, keepdims=True)
        jb = jnp.max(jnp.where(mm == lmax, srow, -1), axis=0, keepdims=True)
        nxt = jnp.sum(jnp.where(srow == jb + 1, x, 0), axis=0, keepdims=True)
        yi = jnp.where(lmax > 0, nxt, -1)
        m_ref[...] = mcur
        acc_ref[...] = jnp.where(srow == i, yi, acc_ref[...])
        return carry

    lax.fori_loop(0, L, step, 0)
    yt = acc_ref[...].T
    y_ref[...] = jnp.concatenate([yt[:512], yt[512:]], axis=1)


def _run_dp(x_pad):
    return pl.pallas_call(
        _dp_body,
        out_shape=jax.ShapeDtypeStruct((512, 128), jnp.int32),
        scratch_shapes=[pltpu.VMEM((LP, B), jnp.int32),
                        pltpu.VMEM((LP, B), jnp.int32)],
    )(x_pad)


_NW = 32
_BPW = BT // _NW
_RPW = B // _NW
_CHUNK = 80
_NCH = _BPW // _CHUNK
_ZR = 80
_CAP = _BPW + _CHUNK + 16


def _bcast_lane(v, lane_idx):
    return lax.gather(
        v, lane_idx[:, None],
        lax.GatherDimensionNumbers(
            offset_dims=(), collapsed_slice_dims=(0,), start_index_map=(0,)),
        (1,), mode=lax.GatherScatterMode.PROMISE_IN_BOUNDS)


def _bcast0(v):
    return _bcast_lane(v, jnp.zeros((16,), jnp.int32))


def _prefix_incl(v, lanes):
    x = v
    for d in (1, 2, 4, 8):
        sh = _bcast_lane(x, jnp.maximum(lanes - d, 0))
        x = x + jnp.where(lanes >= d, sh, 0)
    return x


def _sc_body(table_hbm, t_hbm, out_hbm, t_v, cidx, cdst, grp, lsl, gsem, osem):
    wid = lax.axis_index("s") * 2 + lax.axis_index("c")
    pltpu.sync_copy(t_hbm.at[pl.ds(wid * 16, 16)], t_v)

    def zstore(j, carry):
        lsl[j >> 3, pl.ds((j & 7) * 16, 16)] = jnp.zeros((16,), jnp.float32)
        return carry
    lax.fori_loop(0, 400 * 8, zstore, 0)

    def cgroup(k, n):
        r = k >> 3
        kc = k & 7
        tv = t_v[r, pl.ds(kc * 16, 16)]
        ldbase = (kc >> 2) * 800 + r * L + (kc & 3) * 16
        for j in range(16):
            e = tv[j]
            act = e >= 0

            @pl.when(act)
            def _():
                cidx[pl.ds(n, 16)] = jnp.full((16,), e >> 2, jnp.int32)
                cdst[pl.ds(n, 16)] = jnp.full(
                    (16,), ((e & 3) << 16) | (ldbase + j), jnp.int32)

            n = jnp.where(act, n + 1, n)
        return n

    n = lax.fori_loop(0, 128, cgroup, jnp.int32(0))

    c0 = _bcast0(cidx[pl.ds(0, 16)])
    d0 = _bcast0(cdst[pl.ds(0, 16)])
    for mm in range(_CHUNK // 16):
        cidx[pl.ds(n + mm * 16, 16)] = c0
        cdst[pl.ds(n + mm * 16, 16)] = d0

    nch = (n + (_CHUNK - 1)) // _CHUNK

    def chunk(q, carry):
        g = pltpu.make_async_copy(
            table_hbm.at[cidx.at[pl.ds(q * _CHUNK, _CHUNK)]], grp, gsem)
        g.start()
        g.wait()

        def place(v, carry2):
            dv = cdst[pl.ds(q * _CHUNK + v * 16, 16)]
            for j in range(16):
                d = dv[j]
                sub = (d >> 16) * 32
                slot = d & 0xFFFF
                row = slot >> 2
                col = (slot & 3) * 32
                lsl[row, pl.ds(col, 16)] = grp[v * 16 + j, pl.ds(sub, 16)]
                lsl[row, pl.ds(col + 16, 16)] = grp[v * 16 + j,
                                                    pl.ds(sub + 16, 16)]
            return carry2

        lax.fori_loop(0, _CHUNK // 16, place, 0)
        return carry

    lax.fori_loop(0, nch, chunk, 0)

    o0 = pltpu.make_async_copy(
        lsl.at[pl.ds(0, 200)], out_hbm.at[pl.ds(wid * 200, 200)], osem)
    o1 = pltpu.make_async_copy(
        lsl.at[pl.ds(200, 200)], out_hbm.at[pl.ds(6400 + wid * 200, 200)], osem)
    o0.start()
    o1.start()
    o0.wait()
    o1.wait()


def _run_lookup(emb_weight, t_lin):
    mesh = plsc.VectorSubcoreMesh(core_axis_name="c", subcore_axis_name="s")
    k = functools.partial(
        pl.kernel,
        out_type=jax.ShapeDtypeStruct((12800, 128), jnp.float32),
        mesh=mesh,
        scratch_types=[
            pltpu.VMEM((16, 128), jnp.int32),
            pltpu.VMEM((_CAP,), jnp.int32),
            pltpu.VMEM((_CAP,), jnp.int32),
            pltpu.VMEM((_CHUNK, 128), jnp.float32),
            pltpu.VMEM((400, 128), jnp.float32),
            pltpu.SemaphoreType.DMA,
            pltpu.SemaphoreType.DMA,
        ],
        compiler_params=pltpu.CompilerParams(use_tc_tiling_on_sc=True),
    )(_sc_body)
    return k(lax.optimization_barrier(emb_weight).reshape(250000, 128), t_lin)


def kernel(idx, emb_weight):
    xt = jnp.pad(idx.T, ((0, LP - L), (0, 0)), constant_values=-1)
    y = _run_dp(xt)
    out = _run_lookup(emb_weight, y)
    return out.reshape(BT, C).reshape(B, L, C)
